# Initial kernel scaffold; baseline (speedup 1.0000x reference)
import jax
import jax.numpy as jnp
from jax.experimental import pallas as pl


def kernel(x, edge_index, W1, a_src1, a_dst1, b1, W2, a_src2, a_dst2, b2):
    raise NotImplementedError("write your pallas kernel here")


# speedup vs baseline: 4.3932x; 4.3932x over previous
import dataclasses
import functools

import jax
import jax.numpy as jnp
from jax import lax
from jax.experimental import pallas as pl
from jax.experimental.pallas import tpu as pltpu
from jax.experimental.pallas import tpu_sc as plsc

N = 10000
NP = 10240
TRASH = 10000
EB = 128
BLKS_PER_TILE = 84
EDGES_PER_TILE = BLKS_PER_TILE * EB
EP = 16 * EDGES_PER_TILE
NPASS = 3
ROWS = 3456
ZPT = ROWS // 16
BR = 512
D = 256


def _mm_logits_body(x_ref, w_ref, as_ref, ad_ref, h_ref, asad_ref):
    h = jnp.dot(x_ref[...], w_ref[...], precision=lax.Precision.HIGHEST)
    h_ref[...] = h
    s = jnp.sum(h * as_ref[...], axis=1)
    d = jnp.sum(h * ad_ref[...], axis=1)
    asad_ref[...] = jnp.stack([s, d])


def _tc_layer_in(x_pad, W, a_s, a_d):
    grid = (NP // BR,)
    return pl.pallas_call(
        _mm_logits_body,
        grid=grid,
        in_specs=[
            pl.BlockSpec((BR, D), lambda i: (i, 0)),
            pl.BlockSpec((D, D), lambda i: (0, 0)),
            pl.BlockSpec((1, D), lambda i: (0, 0)),
            pl.BlockSpec((1, D), lambda i: (0, 0)),
        ],
        out_specs=[
            pl.BlockSpec((BR, D), lambda i: (i, 0)),
            pl.BlockSpec((2, BR), lambda i: (0, i)),
        ],
        out_shape=[
            jax.ShapeDtypeStruct((NP, D), jnp.float32),
            jax.ShapeDtypeStruct((2, NP), jnp.float32),
        ],
    )(x_pad, W, a_s.reshape(1, D), a_d.reshape(1, D))


def _mid_body(acc_ref, b_ref, w_ref, as_ref, ad_ref, h_ref, asad_ref):
    acc = acc_ref[...]
    raw = jnp.concatenate([acc[0] + b_ref[0], acc[1] + b_ref[1]], axis=1)
    hid = jnp.where(raw > 0, raw, jnp.exp(jnp.minimum(raw, 0.0)) - 1.0)
    row = pl.program_id(0) * BR + lax.broadcasted_iota(jnp.int32, (BR, 1), 0)
    hid = jnp.where(row < N, hid, 0.0)
    h = jnp.dot(hid, w_ref[...], precision=lax.Precision.HIGHEST)
    h_ref[...] = h
    s = jnp.sum(h * as_ref[...], axis=1)
    d = jnp.sum(h * ad_ref[...], axis=1)
    asad_ref[...] = jnp.stack([s, d])


def _tc_layer_mid(acc, b, W, a_s, a_d):
    grid = (NP // BR,)
    return pl.pallas_call(
        _mid_body,
        grid=grid,
        in_specs=[
            pl.BlockSpec((2, BR, 128), lambda i: (0, i, 0)),
            pl.BlockSpec((2, 1, 128), lambda i: (0, 0, 0)),
            pl.BlockSpec((D, D), lambda i: (0, 0)),
            pl.BlockSpec((1, D), lambda i: (0, 0)),
            pl.BlockSpec((1, D), lambda i: (0, 0)),
        ],
        out_specs=[
            pl.BlockSpec((BR, D), lambda i: (i, 0)),
            pl.BlockSpec((2, BR), lambda i: (0, i)),
        ],
        out_shape=[
            jax.ShapeDtypeStruct((NP, D), jnp.float32),
            jax.ShapeDtypeStruct((2, NP), jnp.float32),
        ],
    )(acc, b.reshape(2, 1, 128), W, a_s.reshape(1, D), a_d.reshape(1, D))


def _out_body(acc_ref, b_ref, o_ref):
    acc = acc_ref[...]
    o_ref[...] = jnp.concatenate([acc[0] + b_ref[0], acc[1] + b_ref[1]], axis=1)


def _tc_layer_out(acc, b):
    grid = (NP // BR,)
    return pl.pallas_call(
        _out_body,
        grid=grid,
        in_specs=[
            pl.BlockSpec((2, BR, 128), lambda i: (0, i, 0)),
            pl.BlockSpec((2, 1, 128), lambda i: (0, 0, 0)),
        ],
        out_specs=pl.BlockSpec((BR, D), lambda i: (i, 0)),
        out_shape=jax.ShapeDtypeStruct((N, D), jnp.float32),
    )(acc, b.reshape(2, 1, 128))


def _sc_agg(h_flat, asad, src, dst):
    mesh = plsc.VectorSubcoreMesh(core_axis_name="c", subcore_axis_name="s")
    cp = pltpu.CompilerParams()
    if "needs_layout_passes" in pltpu.CompilerParams.__dataclass_fields__:
        cp = dataclasses.replace(cp, needs_layout_passes=False)

    @functools.partial(
        pl.kernel,
        mesh=mesh,
        compiler_params=cp,
        out_type=jax.ShapeDtypeStruct((2, NP, 128), jnp.float32),
        scratch_types=[
            pltpu.VMEM((NP,), jnp.float32),
            pltpu.VMEM((NP,), jnp.float32),
            pltpu.VMEM((EB,), jnp.int32),
            pltpu.VMEM((EDGES_PER_TILE,), jnp.int32),
            pltpu.VMEM((EDGES_PER_TILE,), jnp.int32),
            pltpu.VMEM((EDGES_PER_TILE,), jnp.float32),
            pltpu.VMEM((EB,), jnp.int32),
            pltpu.VMEM((EB,), jnp.float32),
            pltpu.VMEM((EB, 128), jnp.float32),
            pltpu.VMEM((80, 128), jnp.float32),
            pltpu.VMEM((80,), jnp.int32),
            pltpu.VMEM((80, 128), jnp.float32),
            pltpu.VMEM((32, 128), jnp.float32),
            pltpu.VMEM((32, 128), jnp.float32),
            pltpu.VMEM_SHARED((ROWS, 128), jnp.float32),
            pltpu.VMEM_SHARED((80, 128), jnp.float32),
        ],
    )
    def kern(h_hbm, asad_hbm, src_hbm, dst_hbm, out_hbm,
             as_v, ad_v, sv, dv_all, gi_all, w_all, li, wm, g, dn_v, idb,
             dchunk, zb, nb, acc_sh, dn_sh):
        c = lax.axis_index("c")
        t = lax.axis_index("s")
        pltpu.sync_copy(asad_hbm.at[0], as_v)
        pltpu.sync_copy(asad_hbm.at[1], ad_v)

        zero16 = jnp.zeros((16,), jnp.float32)

        @pl.loop(0, 32)
        def _(r):
            for k in range(8):
                zb[r, pl.ds(k * 16, 16)] = zero16

        @pl.loop(0, 80)
        def _(r):
            for k in range(8):
                dn_v[r, pl.ds(k * 16, 16)] = zero16

        for k in range(5):
            idb[pl.ds(k * 16, 16)] = lax.iota(jnp.int32, 16) + (k * 16)

        def zero_acc_stripe():
            for o in range(0, ZPT, 32):
                sz = min(32, ZPT - o)
                pltpu.sync_copy(zb.at[pl.ds(0, sz), :],
                                acc_sh.at[pl.ds(t * ZPT + o, sz), :])

        zero_acc_stripe()
        pltpu.sync_copy(zb.at[pl.ds(0, 5), :], dn_sh.at[pl.ds(t * 5, 5), :])
        plsc.subcore_barrier()

        @pl.loop(0, BLKS_PER_TILE)
        def _(blk):
            base = t * EDGES_PER_TILE + blk * EB
            base_l = blk * EB
            pltpu.sync_copy(src_hbm.at[pl.ds(base, EB)], sv)
            pltpu.sync_copy(dst_hbm.at[pl.ds(base, EB)],
                            dv_all.at[pl.ds(base_l, EB)])
            for k in range(EB // 16):
                sll = pl.ds(base_l + k * 16, 16)
                s16 = sv[pl.ds(k * 16, 16)]
                d16 = dv_all[sll]
                gi_all[sll] = s16 * 2 + c
                e = plsc.load_gather(as_v, [s16]) + plsc.load_gather(ad_v, [d16])
                e = jnp.where(e > 0, e, e * 0.2)
                w16 = jnp.exp(e)
                w_all[sll] = w16
                plsc.addupdate_scatter(dn_v, [d16 >> 7, d16 & 127], w16)

        pltpu.sync_copy(dn_v, dn_sh.at[idb], add=True)

        def process_pass(p):
            lo = p * ROWS

            @pl.loop(0, BLKS_PER_TILE)
            def _(blk):
                base_l = blk * EB
                for k in range(EB // 16):
                    sl = pl.ds(k * 16, 16)
                    sll = pl.ds(base_l + k * 16, 16)
                    d16 = dv_all[sll]
                    loc = d16 - lo
                    inr = (d16 >= lo) & (loc < ROWS)
                    li[sl] = jnp.where(inr, loc, 0)
                    wm[sl] = jnp.where(inr, w_all[sll], 0.0)
                pltpu.sync_copy(h_hbm.at[gi_all.at[pl.ds(base_l, EB)]], g)

                @pl.loop(0, EB)
                def _(r):
                    wv = plsc.load_gather(wm, [jnp.full((16,), r, jnp.int32)])
                    for k in range(8):
                        sl = pl.ds(k * 16, 16)
                        g[r, sl] = g[r, sl] * wv

                pltpu.sync_copy(g, acc_sh.at[li], add=True)

        def normalize_and_out(p):
            rpt = min(ROWS, NP - p * ROWS) // 16
            for o in range(0, rpt, 32):
                sz = min(32, rpt - o)

                pltpu.sync_copy(acc_sh.at[pl.ds(t * rpt + o, sz), :],
                                nb.at[pl.ds(0, sz), :])

                @pl.loop(0, sz)
                def _(r):
                    m = p * ROWS + t * rpt + o + r
                    dn16 = plsc.load_gather(
                        dchunk, [jnp.full((16,), m >> 7, jnp.int32),
                                 jnp.full((16,), m & 127, jnp.int32)]) + 1e-16
                    for k in range(8):
                        sl = pl.ds(k * 16, 16)
                        nb[r, sl] = nb[r, sl] / dn16

                pltpu.sync_copy(
                    nb.at[pl.ds(0, sz), :],
                    out_hbm.at[c, pl.ds(p * ROWS + t * rpt + o, sz), :])

        process_pass(0)
        plsc.subcore_barrier()
        pltpu.sync_copy(dn_sh, dchunk)
        for p in range(NPASS):
            if p:
                process_pass(p)
                plsc.subcore_barrier()
            normalize_and_out(p)
            if p + 1 < NPASS:
                plsc.subcore_barrier()
                zero_acc_stripe()
                plsc.subcore_barrier()

    return kern(h_flat, asad, src, dst)


def kernel(x, edge_index, W1, a_src1, a_dst1, b1, W2, a_src2, a_dst2, b2):
    n, e = x.shape[0], edge_index.shape[1]
    loops = jnp.arange(n, dtype=jnp.int32)
    pad = EP - (e + n)
    src = jnp.concatenate([edge_index[0], loops,
                           jnp.zeros((pad,), jnp.int32)])
    dst = jnp.concatenate([edge_index[1], loops,
                           jnp.full((pad,), TRASH, jnp.int32)])
    x_pad = jnp.concatenate([x, jnp.zeros((NP - n, x.shape[1]), x.dtype)])

    h1, asad1 = _tc_layer_in(x_pad, W1, a_src1, a_dst1)
    acc1 = _sc_agg(h1.reshape(2 * NP, 128), asad1, src, dst)
    h2, asad2 = _tc_layer_mid(acc1, b1, W2, a_src2, a_dst2)
    acc2 = _sc_agg(h2.reshape(2 * NP, 128), asad2, src, dst)
    return _tc_layer_out(acc2, b2)


# speedup vs baseline: 7.8165x; 1.7792x over previous
import dataclasses
import functools

import jax
import jax.numpy as jnp
from jax import lax
from jax.experimental import pallas as pl
from jax.experimental.pallas import tpu as pltpu
from jax.experimental.pallas import tpu_sc as plsc

N = 10000
NP = 10240
TRASH = 10000
EB = 128
BLKS_PER_TILE = 84
EDGES_PER_TILE = BLKS_PER_TILE * EB
EP = 16 * EDGES_PER_TILE
NPASS = 3
ROWS = 3456
ZPT = ROWS // 16
ECAP = EDGES_PER_TILE + 2 * EB
BR = 512
D = 256


def _mm_logits_body(x_ref, w_ref, as_ref, ad_ref, h_ref, asad_ref):
    h = jnp.dot(x_ref[...], w_ref[...], precision=lax.Precision.HIGHEST)
    h_ref[...] = h
    s = jnp.sum(h * as_ref[...], axis=1)
    d = jnp.sum(h * ad_ref[...], axis=1)
    asad_ref[...] = jnp.stack([s, d])


def _tc_layer_in(x_pad, W, a_s, a_d):
    grid = (NP // BR,)
    return pl.pallas_call(
        _mm_logits_body,
        grid=grid,
        in_specs=[
            pl.BlockSpec((BR, D), lambda i: (i, 0)),
            pl.BlockSpec((D, D), lambda i: (0, 0)),
            pl.BlockSpec((1, D), lambda i: (0, 0)),
            pl.BlockSpec((1, D), lambda i: (0, 0)),
        ],
        out_specs=[
            pl.BlockSpec((BR, D), lambda i: (i, 0)),
            pl.BlockSpec((2, BR), lambda i: (0, i)),
        ],
        out_shape=[
            jax.ShapeDtypeStruct((NP, D), jnp.float32),
            jax.ShapeDtypeStruct((2, NP), jnp.float32),
        ],
    )(x_pad, W, a_s.reshape(1, D), a_d.reshape(1, D))


def _mid_body(acc_ref, b_ref, w_ref, as_ref, ad_ref, h_ref, asad_ref):
    acc = acc_ref[...]
    raw = jnp.concatenate([acc[0] + b_ref[0], acc[1] + b_ref[1]], axis=1)
    hid = jnp.where(raw > 0, raw, jnp.exp(jnp.minimum(raw, 0.0)) - 1.0)
    row = pl.program_id(0) * BR + lax.broadcasted_iota(jnp.int32, (BR, 1), 0)
    hid = jnp.where(row < N, hid, 0.0)
    h = jnp.dot(hid, w_ref[...], precision=lax.Precision.HIGHEST)
    h_ref[...] = h
    s = jnp.sum(h * as_ref[...], axis=1)
    d = jnp.sum(h * ad_ref[...], axis=1)
    asad_ref[...] = jnp.stack([s, d])


def _tc_layer_mid(acc, b, W, a_s, a_d):
    grid = (NP // BR,)
    return pl.pallas_call(
        _mid_body,
        grid=grid,
        in_specs=[
            pl.BlockSpec((2, BR, 128), lambda i: (0, i, 0)),
            pl.BlockSpec((2, 1, 128), lambda i: (0, 0, 0)),
            pl.BlockSpec((D, D), lambda i: (0, 0)),
            pl.BlockSpec((1, D), lambda i: (0, 0)),
            pl.BlockSpec((1, D), lambda i: (0, 0)),
        ],
        out_specs=[
            pl.BlockSpec((BR, D), lambda i: (i, 0)),
            pl.BlockSpec((2, BR), lambda i: (0, i)),
        ],
        out_shape=[
            jax.ShapeDtypeStruct((NP, D), jnp.float32),
            jax.ShapeDtypeStruct((2, NP), jnp.float32),
        ],
    )(acc, b.reshape(2, 1, 128), W, a_s.reshape(1, D), a_d.reshape(1, D))


def _out_body(acc_ref, b_ref, o_ref):
    acc = acc_ref[...]
    o_ref[...] = jnp.concatenate([acc[0] + b_ref[0], acc[1] + b_ref[1]], axis=1)


def _tc_layer_out(acc, b):
    grid = (NP // BR,)
    return pl.pallas_call(
        _out_body,
        grid=grid,
        in_specs=[
            pl.BlockSpec((2, BR, 128), lambda i: (0, i, 0)),
            pl.BlockSpec((2, 1, 128), lambda i: (0, 0, 0)),
        ],
        out_specs=pl.BlockSpec((BR, D), lambda i: (i, 0)),
        out_shape=jax.ShapeDtypeStruct((N, D), jnp.float32),
    )(acc, b.reshape(2, 1, 128))


def _sc_agg(h_flat, asad, src, dst):
    mesh = plsc.VectorSubcoreMesh(core_axis_name="c", subcore_axis_name="s")
    cp = pltpu.CompilerParams()
    if "needs_layout_passes" in pltpu.CompilerParams.__dataclass_fields__:
        cp = dataclasses.replace(cp, needs_layout_passes=False)

    @functools.partial(
        pl.kernel,
        mesh=mesh,
        compiler_params=cp,
        out_type=jax.ShapeDtypeStruct((2, NP, 128), jnp.float32),
        scratch_types=[
            pltpu.VMEM((NP,), jnp.float32),
            pltpu.VMEM((NP,), jnp.float32),
            pltpu.VMEM((EB,), jnp.int32),
            pltpu.VMEM((EB,), jnp.int32),
            pltpu.VMEM((ECAP,), jnp.int32),
            pltpu.VMEM((ECAP,), jnp.int32),
            pltpu.VMEM((ECAP,), jnp.float32),
            pltpu.VMEM((EB,), jnp.int32),
            pltpu.VMEM((EB,), jnp.int32),
            pltpu.VMEM((EB, 128), jnp.float32),
            pltpu.VMEM((80, 128), jnp.float32),
            pltpu.VMEM((80,), jnp.int32),
            pltpu.VMEM((4, 128), jnp.float32),
            pltpu.VMEM((32, 128), jnp.float32),
            pltpu.VMEM((32, 128), jnp.float32),
            pltpu.VMEM_SHARED((ROWS, 128), jnp.float32),
            pltpu.VMEM_SHARED((80, 128), jnp.float32),
        ],
    )
    def kern(h_hbm, asad_hbm, src_hbm, dst_hbm, out_hbm,
             as_v, ad_v, sv, dv, srcc, dstc, w_l, li, gib, g, dn_v, idb,
             dwin, zb, nb, acc_sh, dn_sh):
        c = lax.axis_index("c")
        t = lax.axis_index("s")
        pltpu.sync_copy(asad_hbm.at[0], as_v)
        pltpu.sync_copy(asad_hbm.at[1], ad_v)

        zero16 = jnp.zeros((16,), jnp.float32)
        zero16i = jnp.zeros((16,), jnp.int32)

        @pl.loop(0, 32)
        def _(r):
            for k in range(8):
                zb[r, pl.ds(k * 16, 16)] = zero16

        @pl.loop(0, 80)
        def _(r):
            for k in range(8):
                dn_v[r, pl.ds(k * 16, 16)] = zero16

        @pl.loop(0, ECAP // 16)
        def _(r):
            sl = pl.ds(r * 16, 16)
            srcc[sl] = zero16i
            dstc[sl] = zero16i
            w_l[sl] = zero16

        for k in range(5):
            idb[pl.ds(k * 16, 16)] = lax.iota(jnp.int32, 16) + (k * 16)

        def zero_acc_stripe():
            for o in range(0, ZPT, 32):
                sz = min(32, ZPT - o)
                pltpu.sync_copy(zb.at[pl.ds(0, sz), :],
                                acc_sh.at[pl.ds(t * ZPT + o, sz), :])

        zero_acc_stripe()
        pltpu.sync_copy(zb.at[pl.ds(0, 5), :], dn_sh.at[pl.ds(t * 5, 5), :])
        plsc.subcore_barrier()

        def count_body(blk, cnt):
            c0, c1, c2 = cnt
            base = t * EDGES_PER_TILE + blk * EB
            pltpu.sync_copy(dst_hbm.at[pl.ds(base, EB)], dv)
            for k in range(EB // 16):
                d16 = dv[pl.ds(k * 16, 16)]
                m1 = (d16 >= ROWS) & (d16 < 2 * ROWS)
                c0 = c0 + jnp.sum((d16 < ROWS).astype(jnp.int32))
                c1 = c1 + jnp.sum(m1.astype(jnp.int32))
                c2 = c2 + jnp.sum((d16 >= 2 * ROWS).astype(jnp.int32))
            return (c0, c1, c2)

        cnt = lax.fori_loop(0, BLKS_PER_TILE, count_body,
                            (jnp.int32(0), jnp.int32(0), jnp.int32(0)))

        def rnd(x):
            return (x + (EB - 1)) & (-EB)

        starts = (jnp.int32(0), rnd(cnt[0]), rnd(cnt[0]) + rnd(cnt[1]))

        def comp_body(blk, offs):
            o0, o1, o2 = offs
            base = t * EDGES_PER_TILE + blk * EB
            pltpu.sync_copy(src_hbm.at[pl.ds(base, EB)], sv)
            pltpu.sync_copy(dst_hbm.at[pl.ds(base, EB)], dv)
            for k in range(EB // 16):
                sl = pl.ds(k * 16, 16)
                s16 = sv[sl]
                d16 = dv[sl]
                e = plsc.load_gather(as_v, [s16]) + plsc.load_gather(ad_v, [d16])
                e = jnp.where(e > 0, e, e * 0.2)
                w16 = jnp.exp(e)
                plsc.addupdate_scatter(dn_v, [d16 >> 7, d16 & 127], w16)
                m0 = d16 < ROWS
                m1 = (d16 >= ROWS) & (d16 < 2 * ROWS)
                m2 = d16 >= 2 * ROWS
                for off, m in ((o0, m0), (o1, m1), (o2, m2)):
                    plsc.store_compressed(srcc.at[pl.ds(off, 16)], s16, mask=m)
                    plsc.store_compressed(dstc.at[pl.ds(off, 16)], d16, mask=m)
                    plsc.store_compressed(w_l.at[pl.ds(off, 16)], w16, mask=m)
                o0 = o0 + jnp.sum(m0.astype(jnp.int32))
                o1 = o1 + jnp.sum(m1.astype(jnp.int32))
                o2 = o2 + jnp.sum(m2.astype(jnp.int32))
            return (o0, o1, o2)

        lax.fori_loop(0, BLKS_PER_TILE, comp_body, starts)

        pltpu.sync_copy(dn_v, dn_sh.at[idb], add=True)

        def process_pass(p):
            lo = p * ROWS
            s_p = starts[p]
            nblk = rnd(cnt[p]) // EB

            def blk_body(blk, carry):
                base = s_p + blk * EB
                for k in range(EB // 16):
                    sl = pl.ds(k * 16, 16)
                    sll = pl.ds(base + k * 16, 16)
                    d16 = dstc[sll]
                    li[sl] = jnp.maximum(d16 - lo, 0)
                    gib[sl] = srcc[sll] * 2 + c
                pltpu.sync_copy(h_hbm.at[gib], g)

                @pl.loop(0, EB)
                def _(r):
                    wv = plsc.load_gather(
                        w_l, [jnp.full((16,), base + r, jnp.int32)])
                    for k in range(8):
                        sl = pl.ds(k * 16, 16)
                        g[r, sl] = g[r, sl] * wv

                pltpu.sync_copy(g, acc_sh.at[li], add=True)
                return carry

            lax.fori_loop(0, nblk, blk_body, jnp.int32(0))

        def normalize_and_out(p):
            rpt = min(ROWS, NP - p * ROWS) // 16
            for o in range(0, rpt, 32):
                sz = min(32, rpt - o)
                m0 = p * ROWS + t * rpt + o
                row0 = jnp.minimum(m0 >> 7, 76)
                pltpu.sync_copy(dn_sh.at[pl.ds(row0, 4), :], dwin)

                pltpu.sync_copy(acc_sh.at[pl.ds(t * rpt + o, sz), :],
                                nb.at[pl.ds(0, sz), :])

                @pl.loop(0, sz)
                def _(r):
                    m = m0 + r
                    dn16 = plsc.load_gather(
                        dwin, [jnp.full((16,), (m >> 7) - row0, jnp.int32),
                               jnp.full((16,), m & 127, jnp.int32)]) + 1e-16
                    for k in range(8):
                        sl = pl.ds(k * 16, 16)
                        nb[r, sl] = nb[r, sl] / dn16

                pltpu.sync_copy(
                    nb.at[pl.ds(0, sz), :],
                    out_hbm.at[c, pl.ds(p * ROWS + t * rpt + o, sz), :])

        process_pass(0)
        plsc.subcore_barrier()
        for p in range(NPASS):
            if p:
                process_pass(p)
                plsc.subcore_barrier()
            normalize_and_out(p)
            if p + 1 < NPASS:
                plsc.subcore_barrier()
                zero_acc_stripe()
                plsc.subcore_barrier()

    return kern(h_flat, asad, src, dst)


def kernel(x, edge_index, W1, a_src1, a_dst1, b1, W2, a_src2, a_dst2, b2):
    n, e = x.shape[0], edge_index.shape[1]
    loops = jnp.arange(n, dtype=jnp.int32)
    pad = EP - (e + n)
    src = jnp.concatenate([edge_index[0], loops,
                           jnp.zeros((pad,), jnp.int32)])
    dst = jnp.concatenate([edge_index[1], loops,
                           jnp.full((pad,), TRASH, jnp.int32)])
    x_pad = jnp.concatenate([x, jnp.zeros((NP - n, x.shape[1]), x.dtype)])

    h1, asad1 = _tc_layer_in(x_pad, W1, a_src1, a_dst1)
    acc1 = _sc_agg(h1.reshape(2 * NP, 128), asad1, src, dst)
    h2, asad2 = _tc_layer_mid(acc1, b1, W2, a_src2, a_dst2)
    acc2 = _sc_agg(h2.reshape(2 * NP, 128), asad2, src, dst)
    return _tc_layer_out(acc2, b2)


# speedup vs baseline: 8.5738x; 1.0969x over previous
import dataclasses
import functools

import jax
import jax.numpy as jnp
from jax import lax
from jax.experimental import pallas as pl
from jax.experimental.pallas import tpu as pltpu
from jax.experimental.pallas import tpu_sc as plsc

N = 10000
NP = 10240
TRASH = 10000
EB = 128
BLKS_PER_TILE = 84
EDGES_PER_TILE = BLKS_PER_TILE * EB
EP = 16 * EDGES_PER_TILE
NPASS = 3
ROWS = 3456
ZPT = ROWS // 16
ECAP = EDGES_PER_TILE + 2 * EB
BR = 512
D = 256


def _mm_logits_body(x_ref, w_ref, as_ref, ad_ref, h_ref, asad_ref):
    h = jnp.dot(x_ref[...], w_ref[...], precision=lax.Precision.HIGHEST)
    h_ref[...] = h
    s = jnp.sum(h * as_ref[...], axis=1)
    d = jnp.sum(h * ad_ref[...], axis=1)
    asad_ref[...] = jnp.stack([s, d])


def _tc_layer_in(x_pad, W, a_s, a_d):
    grid = (NP // BR,)
    return pl.pallas_call(
        _mm_logits_body,
        grid=grid,
        in_specs=[
            pl.BlockSpec((BR, D), lambda i: (i, 0)),
            pl.BlockSpec((D, D), lambda i: (0, 0)),
            pl.BlockSpec((1, D), lambda i: (0, 0)),
            pl.BlockSpec((1, D), lambda i: (0, 0)),
        ],
        out_specs=[
            pl.BlockSpec((BR, D), lambda i: (i, 0)),
            pl.BlockSpec((2, BR), lambda i: (0, i)),
        ],
        out_shape=[
            jax.ShapeDtypeStruct((NP, D), jnp.float32),
            jax.ShapeDtypeStruct((2, NP), jnp.float32),
        ],
    )(x_pad, W, a_s.reshape(1, D), a_d.reshape(1, D))


def _mid_body(acc_ref, b_ref, w_ref, as_ref, ad_ref, h_ref, asad_ref):
    acc = acc_ref[...]
    raw = jnp.concatenate([acc[0] + b_ref[0], acc[1] + b_ref[1]], axis=1)
    hid = jnp.where(raw > 0, raw, jnp.exp(jnp.minimum(raw, 0.0)) - 1.0)
    row = pl.program_id(0) * BR + lax.broadcasted_iota(jnp.int32, (BR, 1), 0)
    hid = jnp.where(row < N, hid, 0.0)
    h = jnp.dot(hid, w_ref[...], precision=lax.Precision.HIGHEST)
    h_ref[...] = h
    s = jnp.sum(h * as_ref[...], axis=1)
    d = jnp.sum(h * ad_ref[...], axis=1)
    asad_ref[...] = jnp.stack([s, d])


def _tc_layer_mid(acc, b, W, a_s, a_d):
    grid = (NP // BR,)
    return pl.pallas_call(
        _mid_body,
        grid=grid,
        in_specs=[
            pl.BlockSpec((2, BR, 128), lambda i: (0, i, 0)),
            pl.BlockSpec((2, 1, 128), lambda i: (0, 0, 0)),
            pl.BlockSpec((D, D), lambda i: (0, 0)),
            pl.BlockSpec((1, D), lambda i: (0, 0)),
            pl.BlockSpec((1, D), lambda i: (0, 0)),
        ],
        out_specs=[
            pl.BlockSpec((BR, D), lambda i: (i, 0)),
            pl.BlockSpec((2, BR), lambda i: (0, i)),
        ],
        out_shape=[
            jax.ShapeDtypeStruct((NP, D), jnp.float32),
            jax.ShapeDtypeStruct((2, NP), jnp.float32),
        ],
    )(acc, b.reshape(2, 1, 128), W, a_s.reshape(1, D), a_d.reshape(1, D))


def _out_body(acc_ref, b_ref, o_ref):
    acc = acc_ref[...]
    o_ref[...] = jnp.concatenate([acc[0] + b_ref[0], acc[1] + b_ref[1]], axis=1)


def _tc_layer_out(acc, b):
    grid = (NP // BR,)
    return pl.pallas_call(
        _out_body,
        grid=grid,
        in_specs=[
            pl.BlockSpec((2, BR, 128), lambda i: (0, i, 0)),
            pl.BlockSpec((2, 1, 128), lambda i: (0, 0, 0)),
        ],
        out_specs=pl.BlockSpec((BR, D), lambda i: (i, 0)),
        out_shape=jax.ShapeDtypeStruct((N, D), jnp.float32),
    )(acc, b.reshape(2, 1, 128))


def _sc_agg(h_flat, asad, src, dst):
    mesh = plsc.VectorSubcoreMesh(core_axis_name="c", subcore_axis_name="s")
    cp = pltpu.CompilerParams()
    if "needs_layout_passes" in pltpu.CompilerParams.__dataclass_fields__:
        cp = dataclasses.replace(cp, needs_layout_passes=False)

    @functools.partial(
        pl.kernel,
        mesh=mesh,
        compiler_params=cp,
        out_type=jax.ShapeDtypeStruct((2, NP, 128), jnp.float32),
        scratch_types=[
            pltpu.VMEM((NP,), jnp.float32),
            pltpu.VMEM((NP,), jnp.float32),
            pltpu.VMEM((EB,), jnp.int32),
            pltpu.VMEM((EB,), jnp.int32),
            pltpu.VMEM((ECAP,), jnp.int32),
            pltpu.VMEM((ECAP,), jnp.int32),
            pltpu.VMEM((ECAP,), jnp.float32),
            pltpu.VMEM((EB,), jnp.int32),
            pltpu.VMEM((EB,), jnp.int32),
            pltpu.VMEM((EB, 128), jnp.float32),
            pltpu.VMEM((80, 128), jnp.float32),
            pltpu.VMEM((80,), jnp.int32),
            pltpu.VMEM((4, 128), jnp.float32),
            pltpu.VMEM((32, 128), jnp.float32),
            pltpu.VMEM((32, 128), jnp.float32),
            pltpu.VMEM_SHARED((ROWS, 128), jnp.float32),
            pltpu.VMEM_SHARED((80, 128), jnp.float32),
        ],
    )
    def kern(h_hbm, asad_hbm, src_hbm, dst_hbm, out_hbm,
             as_v, ad_v, sv, dv, srcc, dstc, w_l, li, gib, g, dn_v, idb,
             dwin, zb, nb, acc_sh, dn_sh):
        c = lax.axis_index("c")
        t = lax.axis_index("s")
        pltpu.sync_copy(asad_hbm.at[0], as_v)
        pltpu.sync_copy(asad_hbm.at[1], ad_v)

        zero16 = jnp.zeros((16,), jnp.float32)
        zero16i = jnp.zeros((16,), jnp.int32)

        @pl.loop(0, 32)
        def _(r):
            for k in range(8):
                zb[r, pl.ds(k * 16, 16)] = zero16

        @pl.loop(0, 80)
        def _(r):
            for k in range(8):
                dn_v[r, pl.ds(k * 16, 16)] = zero16

        @pl.loop(0, ECAP // 16)
        def _(r):
            sl = pl.ds(r * 16, 16)
            srcc[sl] = zero16i
            dstc[sl] = zero16i
            w_l[sl] = zero16

        for k in range(5):
            idb[pl.ds(k * 16, 16)] = lax.iota(jnp.int32, 16) + (k * 16)

        def zero_acc_stripe():
            for o in range(0, ZPT, 32):
                sz = min(32, ZPT - o)
                pltpu.sync_copy(zb.at[pl.ds(0, sz), :],
                                acc_sh.at[pl.ds(t * ZPT + o, sz), :])

        zero_acc_stripe()
        pltpu.sync_copy(zb.at[pl.ds(0, 5), :], dn_sh.at[pl.ds(t * 5, 5), :])
        plsc.subcore_barrier()

        def count_body(blk, cnt):
            c0, c1, c2 = cnt
            base = t * EDGES_PER_TILE + blk * EB
            pltpu.sync_copy(dst_hbm.at[pl.ds(base, EB)], dv)
            for k in range(EB // 16):
                d16 = dv[pl.ds(k * 16, 16)]
                m1 = (d16 >= ROWS) & (d16 < 2 * ROWS)
                c0 = c0 + jnp.sum((d16 < ROWS).astype(jnp.int32))
                c1 = c1 + jnp.sum(m1.astype(jnp.int32))
                c2 = c2 + jnp.sum((d16 >= 2 * ROWS).astype(jnp.int32))
            return (c0, c1, c2)

        cnt = lax.fori_loop(0, BLKS_PER_TILE, count_body,
                            (jnp.int32(0), jnp.int32(0), jnp.int32(0)))

        def rnd(x):
            return (x + (EB - 1)) & (-EB)

        starts = (jnp.int32(0), rnd(cnt[0]), rnd(cnt[0]) + rnd(cnt[1]))

        def comp_body(blk, offs):
            o0, o1, o2 = offs
            base = t * EDGES_PER_TILE + blk * EB
            pltpu.sync_copy(src_hbm.at[pl.ds(base, EB)], sv)
            pltpu.sync_copy(dst_hbm.at[pl.ds(base, EB)], dv)
            for k in range(EB // 16):
                sl = pl.ds(k * 16, 16)
                s16 = sv[sl]
                d16 = dv[sl]
                e = plsc.load_gather(as_v, [s16]) + plsc.load_gather(ad_v, [d16])
                e = jnp.where(e > 0, e, e * 0.2)
                w16 = jnp.exp(e)
                plsc.addupdate_scatter(dn_v, [d16 >> 7, d16 & 127], w16)
                m0 = d16 < ROWS
                m1 = (d16 >= ROWS) & (d16 < 2 * ROWS)
                m2 = d16 >= 2 * ROWS
                for off, m in ((o0, m0), (o1, m1), (o2, m2)):
                    plsc.store_compressed(srcc.at[pl.ds(off, 16)], s16, mask=m)
                    plsc.store_compressed(dstc.at[pl.ds(off, 16)], d16, mask=m)
                    plsc.store_compressed(w_l.at[pl.ds(off, 16)], w16, mask=m)
                o0 = o0 + jnp.sum(m0.astype(jnp.int32))
                o1 = o1 + jnp.sum(m1.astype(jnp.int32))
                o2 = o2 + jnp.sum(m2.astype(jnp.int32))
            return (o0, o1, o2)

        lax.fori_loop(0, BLKS_PER_TILE, comp_body, starts)

        pltpu.sync_copy(dn_v, dn_sh.at[idb], add=True)

        def process_pass(p):
            lo = p * ROWS
            s_p = starts[p]
            nblk = rnd(cnt[p]) // EB

            def blk_body(blk, carry):
                base = s_p + blk * EB
                for k in range(EB // 16):
                    sl = pl.ds(k * 16, 16)
                    sll = pl.ds(base + k * 16, 16)
                    d16 = dstc[sll]
                    li[sl] = jnp.maximum(d16 - lo, 0)
                    gib[sl] = srcc[sll] * 2 + c
                pltpu.sync_copy(h_hbm.at[gib], g)

                @plsc.parallel_loop(0, EB, unroll=4)
                def _(r):
                    wv = plsc.load_gather(
                        w_l, [jnp.full((16,), base + r, jnp.int32)])
                    for k in range(8):
                        sl = pl.ds(k * 16, 16)
                        g[r, sl] = g[r, sl] * wv

                pltpu.sync_copy(g, acc_sh.at[li], add=True)
                return carry

            lax.fori_loop(0, nblk, blk_body, jnp.int32(0))

        def normalize_and_out(p):
            rpt = min(ROWS, NP - p * ROWS) // 16
            for o in range(0, rpt, 32):
                sz = min(32, rpt - o)
                m0 = p * ROWS + t * rpt + o
                row0 = jnp.minimum(m0 >> 7, 76)
                pltpu.sync_copy(dn_sh.at[pl.ds(row0, 4), :], dwin)

                pltpu.sync_copy(acc_sh.at[pl.ds(t * rpt + o, sz), :],
                                nb.at[pl.ds(0, sz), :])

                @plsc.parallel_loop(0, sz, unroll=2)
                def _(r):
                    m = m0 + r
                    dn16 = plsc.load_gather(
                        dwin, [jnp.full((16,), (m >> 7) - row0, jnp.int32),
                               jnp.full((16,), m & 127, jnp.int32)]) + 1e-16
                    for k in range(8):
                        sl = pl.ds(k * 16, 16)
                        nb[r, sl] = nb[r, sl] / dn16

                pltpu.sync_copy(
                    nb.at[pl.ds(0, sz), :],
                    out_hbm.at[c, pl.ds(p * ROWS + t * rpt + o, sz), :])

        process_pass(0)
        plsc.subcore_barrier()
        for p in range(NPASS):
            if p:
                process_pass(p)
                plsc.subcore_barrier()
            normalize_and_out(p)
            if p + 1 < NPASS:
                plsc.subcore_barrier()
                zero_acc_stripe()
                plsc.subcore_barrier()

    return kern(h_flat, asad, src, dst)


def kernel(x, edge_index, W1, a_src1, a_dst1, b1, W2, a_src2, a_dst2, b2):
    n, e = x.shape[0], edge_index.shape[1]
    loops = jnp.arange(n, dtype=jnp.int32)
    pad = EP - (e + n)
    src = jnp.concatenate([edge_index[0], loops,
                           jnp.zeros((pad,), jnp.int32)])
    dst = jnp.concatenate([edge_index[1], loops,
                           jnp.full((pad,), TRASH, jnp.int32)])
    x_pad = jnp.concatenate([x, jnp.zeros((NP - n, x.shape[1]), x.dtype)])

    h1, asad1 = _tc_layer_in(x_pad, W1, a_src1, a_dst1)
    acc1 = _sc_agg(h1.reshape(2 * NP, 128), asad1, src, dst)
    h2, asad2 = _tc_layer_mid(acc1, b1, W2, a_src2, a_dst2)
    acc2 = _sc_agg(h2.reshape(2 * NP, 128), asad2, src, dst)
    return _tc_layer_out(acc2, b2)


# speedup vs baseline: 10.0939x; 1.1773x over previous
import dataclasses
import functools

import jax
import jax.numpy as jnp
from jax import lax
from jax.experimental import pallas as pl
from jax.experimental.pallas import tpu as pltpu
from jax.experimental.pallas import tpu_sc as plsc

N = 10000
NP = 10240
TRASH = 10000
EB = 128
BLKS_PER_TILE = 84
EDGES_PER_TILE = BLKS_PER_TILE * EB
EP = 16 * EDGES_PER_TILE
NPASS = 3
ROWS = 3456
ZPT = ROWS // 16
ECAP = EDGES_PER_TILE + 2 * EB
SB = 1344
NSB = EDGES_PER_TILE // SB
BR = 512
D = 256


def _mm_logits_body(x_ref, w_ref, as_ref, ad_ref, h_ref, asad_ref):
    h = jnp.dot(x_ref[...], w_ref[...], precision=lax.Precision.HIGHEST)
    h_ref[...] = h
    s = jnp.sum(h * as_ref[...], axis=1)
    d = jnp.sum(h * ad_ref[...], axis=1)
    asad_ref[...] = jnp.stack([s, d])


def _tc_layer_in(x_pad, W, a_s, a_d):
    grid = (NP // BR,)
    return pl.pallas_call(
        _mm_logits_body,
        grid=grid,
        in_specs=[
            pl.BlockSpec((BR, D), lambda i: (i, 0)),
            pl.BlockSpec((D, D), lambda i: (0, 0)),
            pl.BlockSpec((1, D), lambda i: (0, 0)),
            pl.BlockSpec((1, D), lambda i: (0, 0)),
        ],
        out_specs=[
            pl.BlockSpec((BR, D), lambda i: (i, 0)),
            pl.BlockSpec((2, BR), lambda i: (0, i)),
        ],
        out_shape=[
            jax.ShapeDtypeStruct((NP, D), jnp.float32),
            jax.ShapeDtypeStruct((2, NP), jnp.float32),
        ],
    )(x_pad, W, a_s.reshape(1, D), a_d.reshape(1, D))


def _mid_body(acc_ref, b_ref, w_ref, as_ref, ad_ref, h_ref, asad_ref):
    acc = acc_ref[...]
    raw = jnp.concatenate([acc[0] + b_ref[0], acc[1] + b_ref[1]], axis=1)
    hid = jnp.where(raw > 0, raw, jnp.exp(jnp.minimum(raw, 0.0)) - 1.0)
    row = pl.program_id(0) * BR + lax.broadcasted_iota(jnp.int32, (BR, 1), 0)
    hid = jnp.where(row < N, hid, 0.0)
    h = jnp.dot(hid, w_ref[...], precision=lax.Precision.HIGHEST)
    h_ref[...] = h
    s = jnp.sum(h * as_ref[...], axis=1)
    d = jnp.sum(h * ad_ref[...], axis=1)
    asad_ref[...] = jnp.stack([s, d])


def _tc_layer_mid(acc, b, W, a_s, a_d):
    grid = (NP // BR,)
    return pl.pallas_call(
        _mid_body,
        grid=grid,
        in_specs=[
            pl.BlockSpec((2, BR, 128), lambda i: (0, i, 0)),
            pl.BlockSpec((2, 1, 128), lambda i: (0, 0, 0)),
            pl.BlockSpec((D, D), lambda i: (0, 0)),
            pl.BlockSpec((1, D), lambda i: (0, 0)),
            pl.BlockSpec((1, D), lambda i: (0, 0)),
        ],
        out_specs=[
            pl.BlockSpec((BR, D), lambda i: (i, 0)),
            pl.BlockSpec((2, BR), lambda i: (0, i)),
        ],
        out_shape=[
            jax.ShapeDtypeStruct((NP, D), jnp.float32),
            jax.ShapeDtypeStruct((2, NP), jnp.float32),
        ],
    )(acc, b.reshape(2, 1, 128), W, a_s.reshape(1, D), a_d.reshape(1, D))


def _out_body(acc_ref, b_ref, o_ref):
    acc = acc_ref[...]
    o_ref[...] = jnp.concatenate([acc[0] + b_ref[0], acc[1] + b_ref[1]], axis=1)


def _tc_layer_out(acc, b):
    grid = (NP // BR,)
    return pl.pallas_call(
        _out_body,
        grid=grid,
        in_specs=[
            pl.BlockSpec((2, BR, 128), lambda i: (0, i, 0)),
            pl.BlockSpec((2, 1, 128), lambda i: (0, 0, 0)),
        ],
        out_specs=pl.BlockSpec((BR, D), lambda i: (i, 0)),
        out_shape=jax.ShapeDtypeStruct((N, D), jnp.float32),
    )(acc, b.reshape(2, 1, 128))


def _sc_agg(h_flat, asad, src, dst):
    mesh = plsc.VectorSubcoreMesh(core_axis_name="c", subcore_axis_name="s")
    cp = pltpu.CompilerParams()
    if "needs_layout_passes" in pltpu.CompilerParams.__dataclass_fields__:
        cp = dataclasses.replace(cp, needs_layout_passes=False)

    @functools.partial(
        pl.kernel,
        mesh=mesh,
        compiler_params=cp,
        out_type=jax.ShapeDtypeStruct((2, NP, 128), jnp.float32),
        scratch_types=[
            pltpu.VMEM((NP,), jnp.float32),
            pltpu.VMEM((NP,), jnp.float32),
            pltpu.VMEM((SB,), jnp.int32),
            pltpu.VMEM((SB,), jnp.int32),
            pltpu.VMEM((ECAP,), jnp.int32),
            pltpu.VMEM((ECAP,), jnp.int32),
            pltpu.VMEM((ECAP,), jnp.float32),
            pltpu.VMEM((EB,), jnp.int32),
            pltpu.VMEM((EB,), jnp.int32),
            pltpu.VMEM((EB, 128), jnp.float32),
            pltpu.VMEM((80, 128), jnp.float32),
            pltpu.VMEM((80,), jnp.int32),
            pltpu.VMEM((4, 128), jnp.float32),
            pltpu.VMEM((32, 128), jnp.float32),
            pltpu.VMEM((32, 128), jnp.float32),
            pltpu.VMEM_SHARED((ROWS, 128), jnp.float32),
            pltpu.VMEM_SHARED((80, 128), jnp.float32),
        ],
    )
    def kern(h_hbm, asad_hbm, src_hbm, dst_hbm, out_hbm,
             as_v, ad_v, sv, dv, srcc, dstc, w_l, li, gib, g, dn_v, idb,
             dwin, zb, nb, acc_sh, dn_sh):
        c = lax.axis_index("c")
        t = lax.axis_index("s")
        pltpu.sync_copy(asad_hbm.at[0], as_v)
        pltpu.sync_copy(asad_hbm.at[1], ad_v)

        zero16 = jnp.zeros((16,), jnp.float32)
        zero16i = jnp.zeros((16,), jnp.int32)

        @pl.loop(0, 32)
        def _(r):
            for k in range(8):
                zb[r, pl.ds(k * 16, 16)] = zero16

        @pl.loop(0, 80)
        def _(r):
            for k in range(8):
                dn_v[r, pl.ds(k * 16, 16)] = zero16

        @pl.loop(0, ECAP // 16)
        def _(r):
            sl = pl.ds(r * 16, 16)
            srcc[sl] = zero16i
            dstc[sl] = zero16i
            w_l[sl] = zero16

        for k in range(5):
            idb[pl.ds(k * 16, 16)] = lax.iota(jnp.int32, 16) + (k * 16)

        def zero_acc_stripe():
            for o in range(0, ZPT, 32):
                sz = min(32, ZPT - o)
                pltpu.sync_copy(zb.at[pl.ds(0, sz), :],
                                acc_sh.at[pl.ds(t * ZPT + o, sz), :])

        zero_acc_stripe()
        pltpu.sync_copy(zb.at[pl.ds(0, 5), :], dn_sh.at[pl.ds(t * 5, 5), :])
        plsc.subcore_barrier()

        def count_body(sb, cnt):
            base = t * EDGES_PER_TILE + sb * SB
            pltpu.sync_copy(dst_hbm.at[pl.ds(base, SB)], dv)

            def inner(k, cnt):
                c0, c1, c2 = cnt
                d16 = dv[pl.ds(k * 16, 16)]
                m1 = (d16 >= ROWS) & (d16 < 2 * ROWS)
                c0 = c0 + jnp.sum((d16 < ROWS).astype(jnp.int32))
                c1 = c1 + jnp.sum(m1.astype(jnp.int32))
                c2 = c2 + jnp.sum((d16 >= 2 * ROWS).astype(jnp.int32))
                return (c0, c1, c2)

            return lax.fori_loop(0, SB // 16, inner, cnt)

        cnt = lax.fori_loop(0, NSB, count_body,
                            (jnp.int32(0), jnp.int32(0), jnp.int32(0)))

        def rnd(x):
            return (x + (EB - 1)) & (-EB)

        starts = (jnp.int32(0), rnd(cnt[0]), rnd(cnt[0]) + rnd(cnt[1]))

        def comp_body(sb, offs):
            base = t * EDGES_PER_TILE + sb * SB
            pltpu.sync_copy(src_hbm.at[pl.ds(base, SB)], sv)
            pltpu.sync_copy(dst_hbm.at[pl.ds(base, SB)], dv)

            def inner(k, offs):
                o0, o1, o2 = offs
                sl = pl.ds(k * 16, 16)
                s16 = sv[sl]
                d16 = dv[sl]
                e = plsc.load_gather(as_v, [s16]) + plsc.load_gather(ad_v, [d16])
                e = jnp.where(e > 0, e, e * 0.2)
                w16 = jnp.exp(e)
                plsc.addupdate_scatter(dn_v, [d16 >> 7, d16 & 127], w16)
                m0 = d16 < ROWS
                m1 = (d16 >= ROWS) & (d16 < 2 * ROWS)
                m2 = d16 >= 2 * ROWS
                for off, m in ((o0, m0), (o1, m1), (o2, m2)):
                    plsc.store_compressed(srcc.at[pl.ds(off, 16)], s16, mask=m)
                    plsc.store_compressed(dstc.at[pl.ds(off, 16)], d16, mask=m)
                    plsc.store_compressed(w_l.at[pl.ds(off, 16)], w16, mask=m)
                o0 = o0 + jnp.sum(m0.astype(jnp.int32))
                o1 = o1 + jnp.sum(m1.astype(jnp.int32))
                o2 = o2 + jnp.sum(m2.astype(jnp.int32))
                return (o0, o1, o2)

            return lax.fori_loop(0, SB // 16, inner, offs)

        lax.fori_loop(0, NSB, comp_body, starts)

        pltpu.sync_copy(dn_v, dn_sh.at[idb], add=True)

        def process_pass(p):
            lo = p * ROWS
            s_p = starts[p]
            nblk = rnd(cnt[p]) // EB

            def blk_body(blk, carry):
                base = s_p + blk * EB
                for k in range(EB // 16):
                    sl = pl.ds(k * 16, 16)
                    sll = pl.ds(base + k * 16, 16)
                    d16 = dstc[sll]
                    li[sl] = jnp.maximum(d16 - lo, 0)
                    gib[sl] = srcc[sll] * 2 + c
                pltpu.sync_copy(h_hbm.at[gib], g)

                @plsc.parallel_loop(0, EB, unroll=4)
                def _(r):
                    wv = plsc.load_gather(
                        w_l, [jnp.full((16,), base + r, jnp.int32)])
                    for k in range(8):
                        sl = pl.ds(k * 16, 16)
                        g[r, sl] = g[r, sl] * wv

                pltpu.sync_copy(g, acc_sh.at[li], add=True)
                return carry

            lax.fori_loop(0, nblk, blk_body, jnp.int32(0))

        def normalize_and_out(p):
            rpt = min(ROWS, NP - p * ROWS) // 16
            for o in range(0, rpt, 32):
                sz = min(32, rpt - o)
                m0 = p * ROWS + t * rpt + o
                row0 = jnp.minimum(m0 >> 7, 76)
                pltpu.sync_copy(dn_sh.at[pl.ds(row0, 4), :], dwin)

                pltpu.sync_copy(acc_sh.at[pl.ds(t * rpt + o, sz), :],
                                nb.at[pl.ds(0, sz), :])

                @plsc.parallel_loop(0, sz, unroll=2)
                def _(r):
                    m = m0 + r
                    dn16 = plsc.load_gather(
                        dwin, [jnp.full((16,), (m >> 7) - row0, jnp.int32),
                               jnp.full((16,), m & 127, jnp.int32)]) + 1e-16
                    for k in range(8):
                        sl = pl.ds(k * 16, 16)
                        nb[r, sl] = nb[r, sl] / dn16

                pltpu.sync_copy(
                    nb.at[pl.ds(0, sz), :],
                    out_hbm.at[c, pl.ds(p * ROWS + t * rpt + o, sz), :])

        process_pass(0)
        plsc.subcore_barrier()
        for p in range(NPASS):
            if p:
                process_pass(p)
                plsc.subcore_barrier()
            normalize_and_out(p)
            if p + 1 < NPASS:
                plsc.subcore_barrier()
                zero_acc_stripe()
                plsc.subcore_barrier()

    return kern(h_flat, asad, src, dst)


def kernel(x, edge_index, W1, a_src1, a_dst1, b1, W2, a_src2, a_dst2, b2):
    n, e = x.shape[0], edge_index.shape[1]
    loops = jnp.arange(n, dtype=jnp.int32)
    pad = EP - (e + n)
    src = jnp.concatenate([edge_index[0], loops,
                           jnp.zeros((pad,), jnp.int32)])
    dst = jnp.concatenate([edge_index[1], loops,
                           jnp.full((pad,), TRASH, jnp.int32)])
    x_pad = jnp.concatenate([x, jnp.zeros((NP - n, x.shape[1]), x.dtype)])

    h1, asad1 = _tc_layer_in(x_pad, W1, a_src1, a_dst1)
    acc1 = _sc_agg(h1.reshape(2 * NP, 128), asad1, src, dst)
    h2, asad2 = _tc_layer_mid(acc1, b1, W2, a_src2, a_dst2)
    acc2 = _sc_agg(h2.reshape(2 * NP, 128), asad2, src, dst)
    return _tc_layer_out(acc2, b2)


# speedup vs baseline: 11.3475x; 1.1242x over previous
import dataclasses
import functools

import jax
import jax.numpy as jnp
from jax import lax
from jax.experimental import pallas as pl
from jax.experimental.pallas import tpu as pltpu
from jax.experimental.pallas import tpu_sc as plsc

N = 10000
NP = 10240
TRASH = 10000
EB = 128
BLKS_PER_TILE = 84
EDGES_PER_TILE = BLKS_PER_TILE * EB
EP = 16 * EDGES_PER_TILE
NPASS = 3
ROWS = 3456
ZPT = ROWS // 16
ECAP = EDGES_PER_TILE + 2 * EB
SB = 1344
NSB = EDGES_PER_TILE // SB
BR = 512
D = 256


def _mm_logits_body(x_ref, w_ref, as_ref, ad_ref, h_ref, asad_ref):
    h = jnp.dot(x_ref[...], w_ref[...], precision=lax.Precision.HIGHEST)
    h_ref[...] = h
    s = jnp.sum(h * as_ref[...], axis=1)
    d = jnp.sum(h * ad_ref[...], axis=1)
    asad_ref[...] = jnp.stack([s, d])


def _tc_layer_in(x_pad, W, a_s, a_d):
    grid = (NP // BR,)
    return pl.pallas_call(
        _mm_logits_body,
        grid=grid,
        in_specs=[
            pl.BlockSpec((BR, D), lambda i: (i, 0)),
            pl.BlockSpec((D, D), lambda i: (0, 0)),
            pl.BlockSpec((1, D), lambda i: (0, 0)),
            pl.BlockSpec((1, D), lambda i: (0, 0)),
        ],
        out_specs=[
            pl.BlockSpec((BR, D), lambda i: (i, 0)),
            pl.BlockSpec((2, BR), lambda i: (0, i)),
        ],
        out_shape=[
            jax.ShapeDtypeStruct((NP, D), jnp.float32),
            jax.ShapeDtypeStruct((2, NP), jnp.float32),
        ],
    )(x_pad, W, a_s.reshape(1, D), a_d.reshape(1, D))


def _mid_body(acc_ref, b_ref, w_ref, as_ref, ad_ref, h_ref, asad_ref):
    acc = acc_ref[...]
    raw = jnp.concatenate([acc[0] + b_ref[0], acc[1] + b_ref[1]], axis=1)
    hid = jnp.where(raw > 0, raw, jnp.exp(jnp.minimum(raw, 0.0)) - 1.0)
    row = pl.program_id(0) * BR + lax.broadcasted_iota(jnp.int32, (BR, 1), 0)
    hid = jnp.where(row < N, hid, 0.0)
    h = jnp.dot(hid, w_ref[...], precision=lax.Precision.HIGHEST)
    h_ref[...] = h
    s = jnp.sum(h * as_ref[...], axis=1)
    d = jnp.sum(h * ad_ref[...], axis=1)
    asad_ref[...] = jnp.stack([s, d])


def _tc_layer_mid(acc, b, W, a_s, a_d):
    grid = (NP // BR,)
    return pl.pallas_call(
        _mid_body,
        grid=grid,
        in_specs=[
            pl.BlockSpec((2, BR, 128), lambda i: (0, i, 0)),
            pl.BlockSpec((2, 1, 128), lambda i: (0, 0, 0)),
            pl.BlockSpec((D, D), lambda i: (0, 0)),
            pl.BlockSpec((1, D), lambda i: (0, 0)),
            pl.BlockSpec((1, D), lambda i: (0, 0)),
        ],
        out_specs=[
            pl.BlockSpec((BR, D), lambda i: (i, 0)),
            pl.BlockSpec((2, BR), lambda i: (0, i)),
        ],
        out_shape=[
            jax.ShapeDtypeStruct((NP, D), jnp.float32),
            jax.ShapeDtypeStruct((2, NP), jnp.float32),
        ],
    )(acc, b.reshape(2, 1, 128), W, a_s.reshape(1, D), a_d.reshape(1, D))


def _out_body(acc_ref, b_ref, o_ref):
    acc = acc_ref[...]
    o_ref[...] = jnp.concatenate([acc[0] + b_ref[0], acc[1] + b_ref[1]], axis=1)


def _tc_layer_out(acc, b):
    grid = (NP // BR,)
    return pl.pallas_call(
        _out_body,
        grid=grid,
        in_specs=[
            pl.BlockSpec((2, BR, 128), lambda i: (0, i, 0)),
            pl.BlockSpec((2, 1, 128), lambda i: (0, 0, 0)),
        ],
        out_specs=pl.BlockSpec((BR, D), lambda i: (i, 0)),
        out_shape=jax.ShapeDtypeStruct((N, D), jnp.float32),
    )(acc, b.reshape(2, 1, 128))


def _sc_agg(h_flat, asad, src, dst):
    mesh = plsc.VectorSubcoreMesh(core_axis_name="c", subcore_axis_name="s")
    cp = pltpu.CompilerParams()
    if "needs_layout_passes" in pltpu.CompilerParams.__dataclass_fields__:
        cp = dataclasses.replace(cp, needs_layout_passes=False)

    @functools.partial(
        pl.kernel,
        mesh=mesh,
        compiler_params=cp,
        out_type=jax.ShapeDtypeStruct((2, NP, 128), jnp.float32),
        scratch_types=[
            pltpu.VMEM((NP,), jnp.float32),
            pltpu.VMEM((NP,), jnp.float32),
            pltpu.VMEM((SB,), jnp.int32),
            pltpu.VMEM((SB,), jnp.int32),
            pltpu.VMEM((ECAP,), jnp.int32),
            pltpu.VMEM((ECAP,), jnp.int32),
            pltpu.VMEM((ECAP,), jnp.float32),
            pltpu.VMEM((64,), jnp.int32),
            pltpu.VMEM((64,), jnp.int32),
            pltpu.VMEM((64,), jnp.int32),
            pltpu.VMEM((64,), jnp.int32),
            pltpu.VMEM((64, 128), jnp.float32),
            pltpu.VMEM((64, 128), jnp.float32),
            pltpu.VMEM((80, 128), jnp.float32),
            pltpu.VMEM((80,), jnp.int32),
            pltpu.VMEM((4, 128), jnp.float32),
            pltpu.VMEM((32, 128), jnp.float32),
            pltpu.VMEM((32, 128), jnp.float32),
            pltpu.VMEM_SHARED((ROWS, 128), jnp.float32),
            pltpu.VMEM_SHARED((80, 128), jnp.float32),
            pltpu.SemaphoreType.DMA,
            pltpu.SemaphoreType.DMA,
        ],
    )
    def kern(h_hbm, asad_hbm, src_hbm, dst_hbm, out_hbm,
             as_v, ad_v, sv, dv, srcc, dstc, w_l, li0, li1, gib0, gib1,
             g0, g1, dn_v, idb, dwin, zb, nb, acc_sh, dn_sh, sem0, sem1):
        c = lax.axis_index("c")
        t = lax.axis_index("s")
        pltpu.sync_copy(asad_hbm.at[0], as_v)
        pltpu.sync_copy(asad_hbm.at[1], ad_v)

        zero16 = jnp.zeros((16,), jnp.float32)
        zero16i = jnp.zeros((16,), jnp.int32)

        @pl.loop(0, 32)
        def _(r):
            for k in range(8):
                zb[r, pl.ds(k * 16, 16)] = zero16

        @pl.loop(0, 80)
        def _(r):
            for k in range(8):
                dn_v[r, pl.ds(k * 16, 16)] = zero16

        @pl.loop(0, ECAP // 16)
        def _(r):
            sl = pl.ds(r * 16, 16)
            srcc[sl] = zero16i
            dstc[sl] = zero16i
            w_l[sl] = zero16

        for k in range(5):
            idb[pl.ds(k * 16, 16)] = lax.iota(jnp.int32, 16) + (k * 16)

        def zero_acc_stripe():
            for o in range(0, ZPT, 32):
                sz = min(32, ZPT - o)
                pltpu.sync_copy(zb.at[pl.ds(0, sz), :],
                                acc_sh.at[pl.ds(t * ZPT + o, sz), :])

        zero_acc_stripe()
        pltpu.sync_copy(zb.at[pl.ds(0, 5), :], dn_sh.at[pl.ds(t * 5, 5), :])
        plsc.subcore_barrier()

        def count_body(sb, cnt):
            base = t * EDGES_PER_TILE + sb * SB
            pltpu.sync_copy(dst_hbm.at[pl.ds(base, SB)], dv)

            def inner(k, cnt):
                c0, c1, c2 = cnt
                d16 = dv[pl.ds(k * 16, 16)]
                m1 = (d16 >= ROWS) & (d16 < 2 * ROWS)
                c0 = c0 + jnp.sum((d16 < ROWS).astype(jnp.int32))
                c1 = c1 + jnp.sum(m1.astype(jnp.int32))
                c2 = c2 + jnp.sum((d16 >= 2 * ROWS).astype(jnp.int32))
                return (c0, c1, c2)

            return lax.fori_loop(0, SB // 16, inner, cnt)

        cnt = lax.fori_loop(0, NSB, count_body,
                            (jnp.int32(0), jnp.int32(0), jnp.int32(0)))

        def rnd(x):
            return (x + (EB - 1)) & (-EB)

        starts = (jnp.int32(0), rnd(cnt[0]), rnd(cnt[0]) + rnd(cnt[1]))

        def comp_body(sb, offs):
            base = t * EDGES_PER_TILE + sb * SB
            pltpu.sync_copy(src_hbm.at[pl.ds(base, SB)], sv)
            pltpu.sync_copy(dst_hbm.at[pl.ds(base, SB)], dv)

            def inner(k, offs):
                o0, o1, o2 = offs
                sl = pl.ds(k * 16, 16)
                s16 = sv[sl]
                d16 = dv[sl]
                e = plsc.load_gather(as_v, [s16]) + plsc.load_gather(ad_v, [d16])
                e = jnp.where(e > 0, e, e * 0.2)
                w16 = jnp.exp(e)
                plsc.addupdate_scatter(dn_v, [d16 >> 7, d16 & 127], w16)
                m0 = d16 < ROWS
                m1 = (d16 >= ROWS) & (d16 < 2 * ROWS)
                m2 = d16 >= 2 * ROWS
                for off, m in ((o0, m0), (o1, m1), (o2, m2)):
                    plsc.store_compressed(srcc.at[pl.ds(off, 16)], s16, mask=m)
                    plsc.store_compressed(dstc.at[pl.ds(off, 16)], d16, mask=m)
                    plsc.store_compressed(w_l.at[pl.ds(off, 16)], w16, mask=m)
                o0 = o0 + jnp.sum(m0.astype(jnp.int32))
                o1 = o1 + jnp.sum(m1.astype(jnp.int32))
                o2 = o2 + jnp.sum(m2.astype(jnp.int32))
                return (o0, o1, o2)

            return lax.fori_loop(0, SB // 16, inner, offs)

        lax.fori_loop(0, NSB, comp_body, starts)

        pltpu.sync_copy(dn_v, dn_sh.at[idb], add=True)

        def process_pass(p):
            lo = p * ROWS
            s_p = starts[p]
            nblk = rnd(cnt[p]) // EB

            def build(base, li_b, gib_b):
                for k in range(4):
                    sl = pl.ds(k * 16, 16)
                    sll = pl.ds(base + k * 16, 16)
                    d16 = dstc[sll]
                    li_b[sl] = jnp.maximum(d16 - lo, 0)
                    gib_b[sl] = srcc[sll] * 2 + c

            def scale(g_b, wbase):
                @plsc.parallel_loop(0, 64, unroll=4)
                def _(r):
                    wv = plsc.load_gather(
                        w_l, [jnp.full((16,), wbase + r, jnp.int32)])
                    for k in range(8):
                        sl = pl.ds(k * 16, 16)
                        g_b[r, sl] = g_b[r, sl] * wv

            build(jnp.minimum(s_p, ECAP - 64), li0, gib0)
            pltpu.async_copy(h_hbm.at[gib0], g0, sem0)

            def blk_body(j, carry):
                base = s_p + j * EB
                build(base + 64, li1, gib1)
                pltpu.async_copy(h_hbm.at[gib1], g1, sem1)
                pltpu.make_async_copy(h_hbm.at[gib0], g0, sem0).wait()
                scale(g0, base)
                pltpu.sync_copy(g0, acc_sh.at[li0], add=True)
                build(jnp.minimum(base + EB, ECAP - 64), li0, gib0)
                pltpu.async_copy(h_hbm.at[gib0], g0, sem0)
                pltpu.make_async_copy(h_hbm.at[gib1], g1, sem1).wait()
                scale(g1, base + 64)
                pltpu.sync_copy(g1, acc_sh.at[li1], add=True)
                return carry

            lax.fori_loop(0, nblk, blk_body, jnp.int32(0))
            pltpu.make_async_copy(h_hbm.at[gib0], g0, sem0).wait()

        def normalize_and_out(p):
            rpt = min(ROWS, NP - p * ROWS) // 16
            for o in range(0, rpt, 32):
                sz = min(32, rpt - o)
                m0 = p * ROWS + t * rpt + o
                row0 = jnp.minimum(m0 >> 7, 76)
                pltpu.sync_copy(dn_sh.at[pl.ds(row0, 4), :], dwin)

                pltpu.sync_copy(acc_sh.at[pl.ds(t * rpt + o, sz), :],
                                nb.at[pl.ds(0, sz), :])

                @plsc.parallel_loop(0, sz, unroll=2)
                def _(r):
                    m = m0 + r
                    dn16 = plsc.load_gather(
                        dwin, [jnp.full((16,), (m >> 7) - row0, jnp.int32),
                               jnp.full((16,), m & 127, jnp.int32)]) + 1e-16
                    for k in range(8):
                        sl = pl.ds(k * 16, 16)
                        nb[r, sl] = nb[r, sl] / dn16

                pltpu.sync_copy(
                    nb.at[pl.ds(0, sz), :],
                    out_hbm.at[c, pl.ds(p * ROWS + t * rpt + o, sz), :])

        process_pass(0)
        plsc.subcore_barrier()
        for p in range(NPASS):
            if p:
                process_pass(p)
                plsc.subcore_barrier()
            normalize_and_out(p)
            if p + 1 < NPASS:
                plsc.subcore_barrier()
                zero_acc_stripe()
                plsc.subcore_barrier()

    return kern(h_flat, asad, src, dst)


def kernel(x, edge_index, W1, a_src1, a_dst1, b1, W2, a_src2, a_dst2, b2):
    n, e = x.shape[0], edge_index.shape[1]
    loops = jnp.arange(n, dtype=jnp.int32)
    pad = EP - (e + n)
    src = jnp.concatenate([edge_index[0], loops,
                           jnp.zeros((pad,), jnp.int32)])
    dst = jnp.concatenate([edge_index[1], loops,
                           jnp.full((pad,), TRASH, jnp.int32)])
    x_pad = jnp.concatenate([x, jnp.zeros((NP - n, x.shape[1]), x.dtype)])

    h1, asad1 = _tc_layer_in(x_pad, W1, a_src1, a_dst1)
    acc1 = _sc_agg(h1.reshape(2 * NP, 128), asad1, src, dst)
    h2, asad2 = _tc_layer_mid(acc1, b1, W2, a_src2, a_dst2)
    acc2 = _sc_agg(h2.reshape(2 * NP, 128), asad2, src, dst)
    return _tc_layer_out(acc2, b2)
